# baseline (speedup 1.0000x reference)
import jax
import jax.numpy as jnp
from jax.experimental import pallas as pl
from jax.experimental.pallas import tpu as pltpu
from jax.experimental.pallas import tpu_sc as plsc

_B = 4096
_D = 4096
_M = 16384
_BLK = 256
_NG = _M // _B


def _interleaved_kernel(x_ref, xout_ref, mem_ref):
    i = pl.program_id(0)
    r = i % _NG

    @pl.when(r == 0)
    def _():
        x = x_ref[...]
        m = jnp.max(x, axis=1, keepdims=True)
        cols = jax.lax.broadcasted_iota(jnp.int32, (_BLK, _D), 1)
        idx = jnp.min(jnp.where(x == m, cols, _D), axis=1, keepdims=True)
        mem_ref[...] = cols == idx
        xout_ref[...] = x

    @pl.when(r != 0)
    def _():
        mem_ref[...] = jnp.zeros((_BLK, _D), jnp.bool_)


def kernel(input, memory):
    grid = _M // _BLK
    _NIN = _B // _BLK

    def mem_map(i):
        q, r = i // _NG, i % _NG
        blk = jnp.where(r == 0, q, _NIN + (_NG - 1) * q + (r - 1))
        return (blk, 0)

    xout, new_mem = pl.pallas_call(
        _interleaved_kernel,
        grid=(grid,),
        in_specs=[pl.BlockSpec((_BLK, _D), lambda i: (i // _NG, 0))],
        out_specs=[
            pl.BlockSpec((_BLK, _D), lambda i: (i // _NG, 0)),
            pl.BlockSpec((_BLK, _D), mem_map),
        ],
        out_shape=[
            jax.ShapeDtypeStruct((_B, _D), input.dtype),
            jax.ShapeDtypeStruct((_M, _D), jnp.bool_),
        ],
        compiler_params=pltpu.CompilerParams(
            dimension_semantics=("arbitrary",),
        ),
    )(input)

    tail = jax.lax.slice(memory, (_B, 0), (_M, _D))
    _CH = 512

    @pl.kernel(
        out_type=jax.ShapeDtypeStruct((_M - _B, _D), jnp.bool_),
        mesh=plsc.ScalarSubcoreMesh(axis_name="core", num_cores=2),
        scratch_types=[
            pltpu.VMEM_SHARED((_CH, _D), jnp.bool_),
            pltpu.SemaphoreType.DMA,
            pltpu.SemaphoreType.DMA,
        ],
    )
    def sc_zero(src_ref, dst_ref, zbuf, lsem, ssem):
        c = jax.lax.axis_index("core")
        half = (_M - _B) // 2
        pltpu.async_copy(src_ref.at[pl.ds(0, _CH), :], zbuf, lsem).wait()
        handles = [
            pltpu.async_copy(
                zbuf, dst_ref.at[pl.ds(c * half + i * _CH, _CH), :], ssem
            )
            for i in range(half // _CH)
        ]
        for h in handles:
            h.wait()

    dummy = sc_zero(tail)
    return (xout, new_mem, dummy)


# speedup vs baseline: 1.6359x; 1.6359x over previous
import jax
import jax.numpy as jnp
from jax.experimental import pallas as pl
from jax.experimental.pallas import tpu as pltpu

_B = 4096
_D = 4096
_M = 16384
_BLK = 256
_NG = _M // _B


def _mem_kernel(x_ref, mem_ref):
    i = pl.program_id(0)
    r = i % _NG

    @pl.when(r == 0)
    def _():
        x = x_ref[...]
        m = jnp.max(x, axis=1, keepdims=True)
        cols = jax.lax.broadcasted_iota(jnp.int32, (_BLK, _D), 1)
        idx = jnp.min(jnp.where(x == m, cols, _D), axis=1, keepdims=True)
        mem_ref[...] = cols == idx

    @pl.when(r != 0)
    def _():
        mem_ref[...] = jnp.zeros((_BLK, _D), jnp.bool_)


def kernel(input, memory):
    grid = _M // _BLK
    _NIN = _B // _BLK

    def mem_map(i):
        q, r = i // _NG, i % _NG
        blk = jnp.where(r == 0, q, _NIN + (_NG - 1) * q + (r - 1))
        return (blk, 0)

    new_mem = pl.pallas_call(
        _mem_kernel,
        grid=(grid,),
        in_specs=[pl.BlockSpec((_BLK, _D), lambda i: (i // _NG, 0))],
        out_specs=pl.BlockSpec((_BLK, _D), mem_map),
        out_shape=jax.ShapeDtypeStruct((_M, _D), jnp.bool_),
        compiler_params=pltpu.CompilerParams(
            dimension_semantics=("arbitrary",),
        ),
    )(input)
    return (input, new_mem)


# speedup vs baseline: 2.0655x; 1.2626x over previous
import jax
import jax.numpy as jnp
from jax.experimental import pallas as pl
from jax.experimental.pallas import tpu as pltpu

_B = 4096
_D = 4096
_M = 16384
_BLK = 256
_NQ = _M // _B


def _mem_kernel(x_ref, xout_ref, mem_ref):
    x = x_ref[...]
    m = jnp.max(x, axis=1, keepdims=True)
    cols = jax.lax.broadcasted_iota(jnp.int32, (_BLK, _D), 1)
    idx = jnp.min(jnp.where(x == m, cols, _D), axis=1, keepdims=True)
    mem_ref[0] = cols == idx
    mem_ref[1] = jnp.zeros((_BLK, _D), jnp.bool_)
    mem_ref[2] = jnp.zeros((_BLK, _D), jnp.bool_)
    mem_ref[3] = jnp.zeros((_BLK, _D), jnp.bool_)
    xout_ref[...] = x


def kernel(input, memory):
    xout, mem4 = pl.pallas_call(
        _mem_kernel,
        grid=(_B // _BLK,),
        in_specs=[pl.BlockSpec((_BLK, _D), lambda q: (q, 0))],
        out_specs=[
            pl.BlockSpec((_BLK, _D), lambda q: (q, 0)),
            pl.BlockSpec((_NQ, _BLK, _D), lambda q: (0, q, 0)),
        ],
        out_shape=[
            jax.ShapeDtypeStruct((_B, _D), input.dtype),
            jax.ShapeDtypeStruct((_NQ, _B, _D), jnp.bool_),
        ],
        compiler_params=pltpu.CompilerParams(
            dimension_semantics=("arbitrary",),
        ),
    )(input)
    return (xout, mem4.reshape(_M, _D))
